# baseline (device time: 77521 ns/iter reference)
import jax
import jax.numpy as jnp
from jax import lax
from jax.experimental import pallas as pl
from jax.experimental.pallas import tpu as pltpu

N_DEV = 32
E_PER = 4
N_BLK = 4


def kernel(x, router_W, route_idx, expert_W):
    n_tok, d_model = x.shape
    e_per, _, d_out = expert_W.shape
    chunk = n_tok // N_DEV
    blk = n_tok // N_BLK
    cpb = N_DEV // N_BLK

    def body(x_ref, rw_ref, idx_ref, w_ref, out_ref,
             x16_ref, w16_ref, stage_ref, comm_ref, gather_ref,
             send_sems, recv1_sems, recv2_sems):
        my = lax.axis_index("i")

        barrier_sem = pltpu.get_barrier_semaphore()
        for k in range(1, N_DEV):
            peer = lax.rem(my + k, N_DEV)
            pl.semaphore_signal(
                barrier_sem, inc=1,
                device_id=(peer,), device_id_type=pl.DeviceIdType.MESH,
            )
        pl.semaphore_wait(barrier_sem, N_DEV - 1)

        x16_ref[:, :] = x_ref[:, :].astype(jnp.bfloat16)
        w16_ref[...] = w_ref[...].astype(jnp.bfloat16)

        p1_sends = []
        for b in range(N_BLK):
            rows = pl.ds(b * blk, blk)
            xb = x16_ref[rows, :]
            mb = idx_ref[rows, :]
            acc = jnp.zeros((blk, d_out), jnp.float32)
            for j in range(E_PER):
                e = my * E_PER + j
                mask = (mb == e).astype(jnp.bfloat16)
                acc = acc + jnp.dot(
                    xb * mask, w16_ref[j], preferred_element_type=jnp.float32,
                )
            stage_ref[rows, :] = acc.astype(jnp.bfloat16)
            for c in range(b * cpb, (b + 1) * cpb):
                rdma = pltpu.make_async_remote_copy(
                    src_ref=stage_ref.at[pl.ds(c * chunk, chunk), :],
                    dst_ref=comm_ref.at[my],
                    send_sem=send_sems.at[c],
                    recv_sem=recv1_sems.at[my],
                    device_id=(c,),
                    device_id_type=pl.DeviceIdType.MESH,
                )
                p1_sends.append((c, rdma))

                @pl.when(my != c)
                def _(rdma=rdma):
                    rdma.start()

        comm_ref[pl.ds(my, 1)] = stage_ref[pl.ds(my * chunk, chunk), :].reshape(
            1, chunk, d_out
        )

        for k in range(1, N_DEV):
            s = lax.rem(my + k, N_DEV)
            recv = pltpu.make_async_remote_copy(
                src_ref=comm_ref.at[s],
                dst_ref=comm_ref.at[s],
                send_sem=send_sems.at[s],
                recv_sem=recv1_sems.at[s],
                device_id=(s,),
                device_id_type=pl.DeviceIdType.MESH,
            )
            recv.wait_recv()

        reduced = jnp.sum(comm_ref[...].astype(jnp.float32), axis=0)
        gather_ref[pl.ds(my, 1)] = reduced.astype(jnp.bfloat16).reshape(
            1, chunk, d_out
        )

        for c, rdma in p1_sends:
            @pl.when(my != c)
            def _(rdma=rdma):
                rdma.wait_send()

        sends2 = []
        for k in range(1, N_DEV):
            t = lax.rem(my + k, N_DEV)
            rdma = pltpu.make_async_remote_copy(
                src_ref=gather_ref.at[my],
                dst_ref=gather_ref.at[my],
                send_sem=send_sems.at[t],
                recv_sem=recv2_sems.at[my],
                device_id=(t,),
                device_id_type=pl.DeviceIdType.MESH,
            )
            rdma.start()
            sends2.append(rdma)

        for k in range(1, N_DEV):
            s = lax.rem(my + k, N_DEV)
            recv = pltpu.make_async_remote_copy(
                src_ref=gather_ref.at[s],
                dst_ref=gather_ref.at[s],
                send_sem=send_sems.at[s],
                recv_sem=recv2_sems.at[s],
                device_id=(s,),
                device_id_type=pl.DeviceIdType.MESH,
            )
            recv.wait_recv()

        out_ref[:, :] = gather_ref[...].reshape(n_tok, d_out).astype(jnp.float32)

        for rdma in sends2:
            rdma.wait_send()

    return pl.pallas_call(
        body,
        out_shape=jax.ShapeDtypeStruct((n_tok, d_out), jnp.float32),
        in_specs=[
            pl.BlockSpec(memory_space=pltpu.VMEM),
            pl.BlockSpec(memory_space=pltpu.VMEM),
            pl.BlockSpec(memory_space=pltpu.VMEM),
            pl.BlockSpec(memory_space=pltpu.VMEM),
        ],
        out_specs=pl.BlockSpec(memory_space=pltpu.VMEM),
        scratch_shapes=[
            pltpu.VMEM((n_tok, d_model), jnp.bfloat16),
            pltpu.VMEM((e_per, d_model, d_out), jnp.bfloat16),
            pltpu.VMEM((n_tok, d_out), jnp.bfloat16),
            pltpu.VMEM((N_DEV, chunk, d_out), jnp.bfloat16),
            pltpu.VMEM((N_DEV, chunk, d_out), jnp.bfloat16),
            pltpu.SemaphoreType.DMA((N_DEV,)),
            pltpu.SemaphoreType.DMA((N_DEV,)),
            pltpu.SemaphoreType.DMA((N_DEV,)),
        ],
        compiler_params=pltpu.CompilerParams(collective_id=0),
    )(x, router_W, route_idx, expert_W)


# device time: 45192 ns/iter; 1.7154x vs baseline; 1.7154x over previous
import jax
import jax.numpy as jnp
from jax import lax
from jax.experimental import pallas as pl
from jax.experimental.pallas import tpu as pltpu

N_DEV = 32
E_PER = 4
N_BLK = 4

import os
ABLATE_P1 = os.environ.get("ABLATE_P1") == "1"
ABLATE_P2 = os.environ.get("ABLATE_P2") == "1"


def kernel(x, router_W, route_idx, expert_W):
    n_tok, d_model = x.shape
    e_per, _, d_out = expert_W.shape
    chunk = n_tok // N_DEV
    blk = n_tok // N_BLK
    cpb = N_DEV // N_BLK

    def body(x_ref, rw_ref, idx_ref, w_ref, out_ref,
             x16_ref, w16_ref, stage_ref, comm_ref, gather_ref,
             send_sems, recv1_sems, recv2_sems):
        my = lax.axis_index("i")

        barrier_sem = pltpu.get_barrier_semaphore()
        for k in range(1, N_DEV):
            peer = lax.rem(my + k, N_DEV)
            pl.semaphore_signal(
                barrier_sem, inc=1,
                device_id=(peer,), device_id_type=pl.DeviceIdType.MESH,
            )
        pl.semaphore_wait(barrier_sem, N_DEV - 1)

        x16_ref[:, :] = x_ref[:, :].astype(jnp.bfloat16)
        w16_ref[...] = w_ref[...].astype(jnp.bfloat16)

        p1_sends = []
        for b in range(N_BLK):
            rows = pl.ds(b * blk, blk)
            xb = x16_ref[rows, :]
            mb = idx_ref[rows, :]
            acc = jnp.zeros((blk, d_out), jnp.float32)
            for j in range(E_PER):
                e = my * E_PER + j
                mask = (mb == e).astype(jnp.bfloat16)
                acc = acc + jnp.dot(
                    xb * mask, w16_ref[j], preferred_element_type=jnp.float32,
                )
            stage_ref[rows, :] = acc.astype(jnp.bfloat16)
            for c in range(b * cpb, (b + 1) * cpb):
                rdma = pltpu.make_async_remote_copy(
                    src_ref=stage_ref.at[pl.ds(c * chunk, chunk), :],
                    dst_ref=comm_ref.at[my],
                    send_sem=send_sems.at[c],
                    recv_sem=recv1_sems.at[my],
                    device_id=(c,),
                    device_id_type=pl.DeviceIdType.MESH,
                )
                p1_sends.append((c, rdma))

                if not ABLATE_P1:
                    @pl.when(my != c)
                    def _(rdma=rdma):
                        rdma.start()

        comm_ref[pl.ds(my, 1)] = stage_ref[pl.ds(my * chunk, chunk), :].reshape(
            1, chunk, d_out
        )

        if not ABLATE_P1:
            for k in range(1, N_DEV):
                s = lax.rem(my + k, N_DEV)
                recv = pltpu.make_async_remote_copy(
                    src_ref=comm_ref.at[s],
                    dst_ref=comm_ref.at[s],
                    send_sem=send_sems.at[s],
                    recv_sem=recv1_sems.at[s],
                    device_id=(s,),
                    device_id_type=pl.DeviceIdType.MESH,
                )
                recv.wait_recv()

        reduced = jnp.sum(comm_ref[...].astype(jnp.float32), axis=0)
        gather_ref[pl.ds(my, 1)] = reduced.astype(jnp.bfloat16).reshape(
            1, chunk, d_out
        )

        if not ABLATE_P1:
            for c, rdma in p1_sends:
                @pl.when(my != c)
                def _(rdma=rdma):
                    rdma.wait_send()

        if not ABLATE_P2:
            sends2 = []
            for k in range(1, N_DEV):
                t = lax.rem(my + k, N_DEV)
                rdma = pltpu.make_async_remote_copy(
                    src_ref=gather_ref.at[my],
                    dst_ref=gather_ref.at[my],
                    send_sem=send_sems.at[t],
                    recv_sem=recv2_sems.at[my],
                    device_id=(t,),
                    device_id_type=pl.DeviceIdType.MESH,
                )
                rdma.start()
                sends2.append(rdma)

            for k in range(1, N_DEV):
                s = lax.rem(my + k, N_DEV)
                recv = pltpu.make_async_remote_copy(
                    src_ref=gather_ref.at[s],
                    dst_ref=gather_ref.at[s],
                    send_sem=send_sems.at[s],
                    recv_sem=recv2_sems.at[s],
                    device_id=(s,),
                    device_id_type=pl.DeviceIdType.MESH,
                )
                recv.wait_recv()

        out_ref[:, :] = gather_ref[...].reshape(n_tok, d_out).astype(jnp.float32)

        if not ABLATE_P2:
            for rdma in sends2:
                rdma.wait_send()

    return pl.pallas_call(
        body,
        out_shape=jax.ShapeDtypeStruct((n_tok, d_out), jnp.float32),
        in_specs=[
            pl.BlockSpec(memory_space=pltpu.VMEM),
            pl.BlockSpec(memory_space=pltpu.VMEM),
            pl.BlockSpec(memory_space=pltpu.VMEM),
            pl.BlockSpec(memory_space=pltpu.VMEM),
        ],
        out_specs=pl.BlockSpec(memory_space=pltpu.VMEM),
        scratch_shapes=[
            pltpu.VMEM((n_tok, d_model), jnp.bfloat16),
            pltpu.VMEM((e_per, d_model, d_out), jnp.bfloat16),
            pltpu.VMEM((n_tok, d_out), jnp.bfloat16),
            pltpu.VMEM((N_DEV, chunk, d_out), jnp.bfloat16),
            pltpu.VMEM((N_DEV, chunk, d_out), jnp.bfloat16),
            pltpu.SemaphoreType.DMA((N_DEV,)),
            pltpu.SemaphoreType.DMA((N_DEV,)),
            pltpu.SemaphoreType.DMA((N_DEV,)),
        ],
        compiler_params=pltpu.CompilerParams(collective_id=0),
    )(x, router_W, route_idx, expert_W)
